# Initial kernel scaffold; baseline (speedup 1.0000x reference)
import jax
import jax.numpy as jnp
from jax.experimental import pallas as pl


def kernel(token_ids, weight):
    raise NotImplementedError("write your pallas kernel here")


# speedup vs baseline: 1.5077x; 1.5077x over previous
import functools

import jax
import jax.numpy as jnp
from jax import lax
from jax.experimental import pallas as pl
from jax.experimental.pallas import tpu as pltpu
from jax.experimental.pallas import tpu_sc as plsc

NUM_CORES = 2
NUM_SUBCORES = 16
NUM_WORKERS = NUM_CORES * NUM_SUBCORES
CHUNK = 2048


@functools.partial(jax.jit, static_argnums=(2, 3))
def _sc_embed(flat_ids, weight, B, D):
    b_per_w = B // NUM_WORKERS
    n_chunks = b_per_w // CHUNK
    mesh = plsc.VectorSubcoreMesh(
        core_axis_name="c", subcore_axis_name="s",
        num_cores=NUM_CORES, num_subcores=NUM_SUBCORES)

    @functools.partial(
        pl.kernel,
        out_type=jax.ShapeDtypeStruct((B, D), weight.dtype),
        mesh=mesh,
        scratch_types=[
            pltpu.VMEM((CHUNK,), jnp.int32),
            pltpu.VMEM((CHUNK, D), weight.dtype),
            pltpu.SemaphoreType.DMA,
        ],
        compiler_params=pltpu.CompilerParams(use_tc_tiling_on_sc=False),
    )
    def k(idx_hbm, table_hbm, out_hbm, idx_v, rows_v, sem):
        wid = lax.axis_index("s") * NUM_CORES + lax.axis_index("c")
        base = wid * b_per_w

        def body(i, carry):
            off = base + i * CHUNK
            pltpu.sync_copy(idx_hbm.at[pl.ds(off, CHUNK)], idx_v)
            pltpu.async_copy(table_hbm.at[idx_v], rows_v, sem).wait()
            pltpu.sync_copy(rows_v, out_hbm.at[pl.ds(off, CHUNK)])
            return carry

        lax.fori_loop(0, n_chunks, body, 0)

    return k(flat_ids, weight)


def kernel(token_ids, weight):
    N, S = token_ids.shape
    B = N * S
    D = weight.shape[1]
    flat = token_ids.reshape(B).astype(jnp.int32)
    out = _sc_embed(flat, weight, B, D)
    return out.reshape(N, S, D)


# speedup vs baseline: 1.5125x; 1.0032x over previous
import functools

import jax
import jax.numpy as jnp
from jax import lax
from jax.experimental import pallas as pl
from jax.experimental.pallas import tpu as pltpu
from jax.experimental.pallas import tpu_sc as plsc

NUM_CORES = 2
NUM_SUBCORES = 16
NUM_WORKERS = NUM_CORES * NUM_SUBCORES
CHUNK = 1280


@functools.partial(jax.jit, static_argnums=(2, 3))
def _sc_embed(flat_ids, weight, B, D):
    b_per_w = B // NUM_WORKERS
    n_chunks = b_per_w // CHUNK
    mesh = plsc.VectorSubcoreMesh(
        core_axis_name="c", subcore_axis_name="s",
        num_cores=NUM_CORES, num_subcores=NUM_SUBCORES)

    @functools.partial(
        pl.kernel,
        out_type=jax.ShapeDtypeStruct((B, D), weight.dtype),
        mesh=mesh,
        scratch_types=[
            pltpu.VMEM((CHUNK,), jnp.int32),
            pltpu.VMEM((CHUNK,), jnp.int32),
            pltpu.VMEM((CHUNK, D), weight.dtype),
            pltpu.VMEM((CHUNK, D), weight.dtype),
            pltpu.SemaphoreType.DMA,
            pltpu.SemaphoreType.DMA,
            pltpu.SemaphoreType.DMA,
            pltpu.SemaphoreType.DMA,
            pltpu.SemaphoreType.DMA,
            pltpu.SemaphoreType.DMA,
        ],
        compiler_params=pltpu.CompilerParams(use_tc_tiling_on_sc=False),
    )
    def k(idx_hbm, table_hbm, out_hbm, idx0, idx1, rows0, rows1,
          ls0, ls1, gs0, gs1, ws0, ws1):
        wid = lax.axis_index("s") * NUM_CORES + lax.axis_index("c")
        base = wid * b_per_w
        idx = [idx0, idx1]
        rows = [rows0, rows1]
        lsem = [ls0, ls1]
        gsem = [gs0, gs1]
        wsem = [ws0, ws1]

        def load(i):
            b = i % 2
            return pltpu.async_copy(
                idx_hbm.at[pl.ds(base + i * CHUNK, CHUNK)], idx[b], lsem[b])

        def gather(i):
            b = i % 2
            return pltpu.async_copy(table_hbm.at[idx[b]], rows[b], gsem[b])

        def writeback(i):
            b = i % 2
            return pltpu.async_copy(
                rows[b], out_hbm.at[pl.ds(base + i * CHUNK, CHUNK)], wsem[b])

        loads = [None] * n_chunks
        gathers = [None] * n_chunks
        writes = [None] * n_chunks
        loads[0] = load(0)
        if n_chunks > 1:
            loads[1] = load(1)
        for i in range(n_chunks):
            loads[i].wait()
            if i >= 2:
                writes[i - 2].wait()
            gathers[i] = gather(i)
            if i >= 1:
                gathers[i - 1].wait()
                writes[i - 1] = writeback(i - 1)
                if i + 1 < n_chunks:
                    loads[i + 1] = load(i + 1)
        gathers[n_chunks - 1].wait()
        writes[n_chunks - 1] = writeback(n_chunks - 1)
        if n_chunks > 1:
            writes[n_chunks - 2].wait()
        writes[n_chunks - 1].wait()

    return k(flat_ids, weight)


def kernel(token_ids, weight):
    N, S = token_ids.shape
    B = N * S
    D = weight.shape[1]
    flat = token_ids.reshape(B).astype(jnp.int32)
    out = _sc_embed(flat, weight, B, D)
    return out.reshape(N, S, D)


# speedup vs baseline: 1.5138x; 1.0008x over previous
import functools

import jax
import jax.numpy as jnp
from jax import lax
from jax.experimental import pallas as pl
from jax.experimental.pallas import tpu as pltpu
from jax.experimental.pallas import tpu_sc as plsc

NUM_CORES = 2
NUM_SUBCORES = 16
NUM_WORKERS = NUM_CORES * NUM_SUBCORES
CHUNK = 1280


@functools.partial(jax.jit, static_argnums=(2, 3))
def _sc_embed(flat_ids, weight, B, D):
    b_per_w = B // NUM_WORKERS
    n_chunks = b_per_w // CHUNK
    mesh = plsc.VectorSubcoreMesh(
        core_axis_name="c", subcore_axis_name="s",
        num_cores=NUM_CORES, num_subcores=NUM_SUBCORES)

    @functools.partial(
        pl.kernel,
        out_type=jax.ShapeDtypeStruct((B, D), weight.dtype),
        mesh=mesh,
        scratch_types=[
            pltpu.VMEM((CHUNK,), jnp.int32),
            pltpu.VMEM((CHUNK,), jnp.int32),
            pltpu.VMEM((CHUNK, D), weight.dtype),
            pltpu.VMEM((CHUNK, D), weight.dtype),
            pltpu.SemaphoreType.DMA,
            pltpu.SemaphoreType.DMA,
            pltpu.SemaphoreType.DMA,
            pltpu.SemaphoreType.DMA,
            pltpu.SemaphoreType.DMA,
            pltpu.SemaphoreType.DMA,
        ],
        compiler_params=pltpu.CompilerParams(use_tc_tiling_on_sc=False),
    )
    def k(idx_hbm, table_hbm, out_hbm, idx0, idx1, rows0, rows1,
          ls0, ls1, gs0, gs1, ws0, ws1):
        wid = lax.axis_index("s") * NUM_CORES + lax.axis_index("c")
        base = wid * b_per_w
        idx = [idx0, idx1]
        rows = [rows0, rows1]
        lsem = [ls0, ls1]
        gsem = [gs0, gs1]
        wsem = [ws0, ws1]

        def load(i):
            b = i % 2
            return pltpu.async_copy(
                idx_hbm.at[pl.ds(base + i * CHUNK, CHUNK)], idx[b], lsem[b])

        def gather(i):
            b = i % 2
            return pltpu.async_copy(table_hbm.at[idx[b]], rows[b], gsem[b])

        def writeback(i):
            b = i % 2
            return pltpu.async_copy(
                rows[b], out_hbm.at[pl.ds(base + i * CHUNK, CHUNK)], wsem[b])

        loads = [None] * n_chunks
        gathers = [None] * n_chunks
        writes = [None] * n_chunks
        loads[0] = load(0)
        if n_chunks > 1:
            loads[1] = load(1)
        for i in range(n_chunks):
            loads[i].wait()
            if i >= 2:
                writes[i - 2].wait()
            gathers[i] = gather(i)
            if i >= 1:
                gathers[i - 1].wait()
                writes[i - 1] = writeback(i - 1)
                if i + 1 < n_chunks:
                    loads[i + 1] = load(i + 1)
        gathers[n_chunks - 1].wait()
        writes[n_chunks - 1] = writeback(n_chunks - 1)
        if n_chunks > 1:
            writes[n_chunks - 2].wait()
        writes[n_chunks - 1].wait()

    return k(flat_ids, weight)


def kernel(token_ids, weight):
    N, S = token_ids.shape
    B = N * S
    D = weight.shape[1]
    flat = token_ids.reshape(B).astype(jnp.int32)
    w_t = jax.lax.optimization_barrier(weight.T)
    w_row = w_t.T
    out = _sc_embed(flat, w_row, B, D)
    return out.reshape(N, S, D)
